# speedup vs baseline: 14.2814x; 14.2814x over previous
import jax
import jax.numpy as jnp
from jax import lax
from jax.experimental import pallas as pl

_N = 20000
_T = 100
_B = 8
_V0 = 0.1
_V1 = 0.2
_THR = 0.5


def _body(pri_ref, gt_ref, lab_ref, loc_ref, labout_ref):
    pcx = pri_ref[0:1, :]
    pcy = pri_ref[1:2, :]
    pw = pri_ref[2:3, :]
    ph = pri_ref[3:4, :]
    px1 = pcx - pw * 0.5
    py1 = pcy - ph * 0.5
    px2 = pcx + pw * 0.5
    py2 = pcy + ph * 0.5
    parea = pw * ph

    gx1 = gt_ref[0, :, 0:1]
    gy1 = gt_ref[0, :, 1:2]
    gx2 = gt_ref[0, :, 2:3]
    gy2 = gt_ref[0, :, 3:4]
    gw = gx2 - gx1
    gh = gy2 - gy1
    garea = gw * gh

    ix1 = jnp.maximum(px1, gx1)
    iy1 = jnp.maximum(py1, gy1)
    ix2 = jnp.minimum(px2, gx2)
    iy2 = jnp.minimum(py2, gy2)
    iw = jnp.maximum(ix2 - ix1, 0.0)
    ih = jnp.maximum(iy2 - iy1, 0.0)
    inter = iw * ih
    iou = inter / (parea + garea - inter)

    trow = lax.broadcasted_iota(jnp.int32, (_T, _N), 0)
    ncol = lax.broadcasted_iota(jnp.int32, (_T, _N), 1)

    mv = jnp.max(iou, axis=0, keepdims=True)
    mt = jnp.min(jnp.where(iou == mv, trow, _T), axis=0, keepdims=True)

    bv = jnp.max(iou, axis=1, keepdims=True)
    bp = jnp.min(jnp.where(iou == bv, ncol, _N), axis=1, keepdims=True)

    hit = ncol == bp
    t_at = jnp.max(jnp.where(hit, trow, -1), axis=0, keepdims=True)
    forced = t_at >= 0
    mt = jnp.where(forced, t_at, mt)
    mv = jnp.where(forced, 2.0, mv)

    onehot = trow == mt

    def gsel(col):
        return jnp.sum(jnp.where(onehot, col, 0.0), axis=0, keepdims=True)

    bcx = gsel((gx1 + gx2) * 0.5)
    bcy = gsel((gy1 + gy2) * 0.5)
    bw = gsel(gw)
    bh = gsel(gh)
    labv = gsel(lab_ref[0, :, 0:1])
    labv = jnp.where(mv < _THR, 0.0, labv)

    loc_ref[0, 0:1, :] = (bcx - pcx) / pw * (1.0 / _V0)
    loc_ref[0, 1:2, :] = (bcy - pcy) / ph * (1.0 / _V0)
    loc_ref[0, 2:3, :] = jnp.log(bw / pw) * (1.0 / _V1)
    loc_ref[0, 3:4, :] = jnp.log(bh / ph) * (1.0 / _V1)
    labout_ref[0, 0:1, :] = labv.astype(jnp.int32)


def kernel(priors_xywha, gt_boxes, gt_labels):
    priors_t = priors_xywha.T
    labf = gt_labels.astype(jnp.float32).reshape(_B, _T, 1)
    loc_t, lab3 = pl.pallas_call(
        _body,
        grid=(_B,),
        in_specs=[
            pl.BlockSpec((4, _N), lambda b: (0, 0)),
            pl.BlockSpec((1, _T, 4), lambda b: (b, 0, 0)),
            pl.BlockSpec((1, _T, 1), lambda b: (b, 0, 0)),
        ],
        out_specs=[
            pl.BlockSpec((1, 4, _N), lambda b: (b, 0, 0)),
            pl.BlockSpec((1, 1, _N), lambda b: (b, 0, 0)),
        ],
        out_shape=[
            jax.ShapeDtypeStruct((_B, 4, _N), jnp.float32),
            jax.ShapeDtypeStruct((_B, 1, _N), jnp.int32),
        ],
    )(priors_t, gt_boxes, labf)
    return jnp.transpose(loc_t, (0, 2, 1)), lab3.reshape(_B, _N)
